# Initial kernel scaffold; baseline (speedup 1.0000x reference)
import jax
import jax.numpy as jnp
from jax.experimental import pallas as pl


def kernel(q, k, v, means, mem_key, mem_value):
    raise NotImplementedError("write your pallas kernel here")


# speedup vs baseline: 5.1354x; 5.1354x over previous
import jax
import jax.numpy as jnp
from jax.experimental import pallas as pl

H = 12
T = 2048
D = 64
NC = 2
BQ = 512
SCALE = D ** -0.5
NEG = -1e9
EPS = 1e-6
COMMITMENT = 0.0001
_PREC = jax.lax.Precision.HIGHEST


def _attn_kernel(q_ref, k_ref, v_ref, means_ref, memk_ref, memv_ref,
                 out_ref, loss_ref):
    h = pl.program_id(0)
    i = pl.program_id(1)
    qb = q_ref[0]
    kf = k_ref[0]
    vf = v_ref[0]

    s = jax.lax.dot_general(qb, kf, (((1,), (1,)), ((), ())),
                            preferred_element_type=jnp.float32,
                            precision=_PREC) * SCALE
    rows = jax.lax.broadcasted_iota(jnp.int32, (BQ, T), 0) + i * BQ
    cols = jax.lax.broadcasted_iota(jnp.int32, (BQ, T), 1)
    s = jnp.where(rows == cols, NEG, s)

    memk = memk_ref[0]
    mc = jax.lax.dot_general(qb, memk, (((1,), (1,)), ((), ())),
                             preferred_element_type=jnp.float32,
                             precision=_PREC) * SCALE
    m = jnp.maximum(jnp.max(s, axis=1), jnp.max(mc, axis=1))
    e = jnp.exp(s - m[:, None])
    z = jnp.sum(e, axis=1)
    n = jax.lax.dot_general(e, vf, (((1,), (0,)), ((), ())),
                            preferred_element_type=jnp.float32,
                            precision=_PREC)
    em = jnp.exp(mc - m[:, None])
    memv = memv_ref[0]
    acc = jnp.zeros_like(n)
    for c in range(NC):
        acc = acc + (n + em[:, c:c + 1] * memv[c][None, :]) \
            / (z + em[:, c])[:, None]
    out_ref[0] = acc * (1.0 / (NC + 1e-5))

    means = means_ref[0]
    nrm = jnp.sqrt(jnp.sum(qb * qb, axis=1))
    xn = qb / (nrm + EPS)[:, None]
    x2 = jnp.sum(xn * xn, axis=1)
    m2 = jnp.sum(means * means, axis=1)
    xm = jax.lax.dot_general(xn, means, (((1,), (1,)), ((), ())),
                             preferred_element_type=jnp.float32,
                             precision=_PREC)
    d2 = jnp.maximum(x2[:, None] + m2[None, :] - 2.0 * xm, 0.0)
    pick0 = d2[:, 0] <= d2[:, 1]
    routed = jnp.where(pick0[:, None], means[0][None, :], means[1][None, :])
    part = (jnp.sum((xn - routed) ** 2)
            * (COMMITMENT / (H * T * D))).reshape(1, 1)

    @pl.when(jnp.logical_and(h == 0, i == 0))
    def _init():
        loss_ref[...] = jnp.zeros((1, 1), jnp.float32)

    loss_ref[...] += part


def kernel(q, k, v, means, mem_key, mem_value):
    b = q.shape[0]
    qh = q.reshape(H, T, D)
    kh = k.reshape(H, T, D)
    vh = v.reshape(H, T, D)
    memk = mem_key.reshape(H, NC, D)
    memv = mem_value.reshape(H, NC, D)
    out, loss = pl.pallas_call(
        _attn_kernel,
        grid=(H, T // BQ),
        in_specs=[
            pl.BlockSpec((1, BQ, D), lambda h, i: (h, i, 0)),
            pl.BlockSpec((1, T, D), lambda h, i: (h, 0, 0)),
            pl.BlockSpec((1, T, D), lambda h, i: (h, 0, 0)),
            pl.BlockSpec((1, NC, D), lambda h, i: (h, 0, 0)),
            pl.BlockSpec((1, NC, D), lambda h, i: (h, 0, 0)),
            pl.BlockSpec((1, NC, D), lambda h, i: (h, 0, 0)),
        ],
        out_specs=[
            pl.BlockSpec((1, BQ, D), lambda h, i: (h, i, 0)),
            pl.BlockSpec((1, 1), lambda h, i: (0, 0)),
        ],
        out_shape=[
            jax.ShapeDtypeStruct((H, T, D), jnp.float32),
            jax.ShapeDtypeStruct((1, 1), jnp.float32),
        ],
    )(qh, kh, vh, means, memk, memv)
    return out.reshape(b, H, T, D), loss[0, 0]


# speedup vs baseline: 17.5015x; 3.4080x over previous
import jax
import jax.numpy as jnp
from jax.experimental import pallas as pl

H = 12
T = 2048
D = 64
NC = 2
BQ = 512
SCALE = D ** -0.5
NEG = -1e9
EPS = 1e-6
COMMITMENT = 0.0001
_PREC = jax.lax.Precision.DEFAULT


def _attn_kernel(q_ref, k_ref, v_ref, means_ref, memk_ref, memv_ref,
                 out_ref, loss_ref):
    h = pl.program_id(0)
    i = pl.program_id(1)
    qb = q_ref[0]
    kf = k_ref[0]
    vf = v_ref[0]

    s = jax.lax.dot_general(qb, kf, (((1,), (1,)), ((), ())),
                            preferred_element_type=jnp.float32,
                            precision=_PREC) * SCALE
    rows = jax.lax.broadcasted_iota(jnp.int32, (BQ, T), 0) + i * BQ
    cols = jax.lax.broadcasted_iota(jnp.int32, (BQ, T), 1)
    s = jnp.where(rows == cols, NEG, s)

    memk = memk_ref[0]
    mc = jax.lax.dot_general(qb, memk, (((1,), (1,)), ((), ())),
                             preferred_element_type=jnp.float32,
                             precision=_PREC) * SCALE
    m = jnp.maximum(jnp.max(s, axis=1), jnp.max(mc, axis=1))
    e = jnp.exp(s - m[:, None])
    z = jnp.sum(e, axis=1)
    n = jax.lax.dot_general(e, vf, (((1,), (0,)), ((), ())),
                            preferred_element_type=jnp.float32,
                            precision=_PREC)
    em = jnp.exp(mc - m[:, None])
    memv = memv_ref[0]
    acc = jnp.zeros_like(n)
    for c in range(NC):
        acc = acc + (n + em[:, c:c + 1] * memv[c][None, :]) \
            / (z + em[:, c])[:, None]
    out_ref[0] = acc * (1.0 / (NC + 1e-5))

    means = means_ref[0]
    nrm = jnp.sqrt(jnp.sum(qb * qb, axis=1))
    xn = qb / (nrm + EPS)[:, None]
    x2 = jnp.sum(xn * xn, axis=1)
    m2 = jnp.sum(means * means, axis=1)
    xm = jax.lax.dot_general(xn, means, (((1,), (1,)), ((), ())),
                             preferred_element_type=jnp.float32,
                             precision=_PREC)
    d2 = jnp.maximum(x2[:, None] + m2[None, :] - 2.0 * xm, 0.0)
    pick0 = d2[:, 0] <= d2[:, 1]
    routed = jnp.where(pick0[:, None], means[0][None, :], means[1][None, :])
    part = (jnp.sum((xn - routed) ** 2)
            * (COMMITMENT / (H * T * D))).reshape(1, 1)

    @pl.when(jnp.logical_and(h == 0, i == 0))
    def _init():
        loss_ref[...] = jnp.zeros((1, 1), jnp.float32)

    loss_ref[...] += part


def kernel(q, k, v, means, mem_key, mem_value):
    b = q.shape[0]
    qh = q.reshape(H, T, D)
    kh = k.reshape(H, T, D)
    vh = v.reshape(H, T, D)
    memk = mem_key.reshape(H, NC, D)
    memv = mem_value.reshape(H, NC, D)
    out, loss = pl.pallas_call(
        _attn_kernel,
        grid=(H, T // BQ),
        in_specs=[
            pl.BlockSpec((1, BQ, D), lambda h, i: (h, i, 0)),
            pl.BlockSpec((1, T, D), lambda h, i: (h, 0, 0)),
            pl.BlockSpec((1, T, D), lambda h, i: (h, 0, 0)),
            pl.BlockSpec((1, NC, D), lambda h, i: (h, 0, 0)),
            pl.BlockSpec((1, NC, D), lambda h, i: (h, 0, 0)),
            pl.BlockSpec((1, NC, D), lambda h, i: (h, 0, 0)),
        ],
        out_specs=[
            pl.BlockSpec((1, BQ, D), lambda h, i: (h, i, 0)),
            pl.BlockSpec((1, 1), lambda h, i: (0, 0)),
        ],
        out_shape=[
            jax.ShapeDtypeStruct((H, T, D), jnp.float32),
            jax.ShapeDtypeStruct((1, 1), jnp.float32),
        ],
    )(qh, kh, vh, means, memk, memv)
    return out.reshape(b, H, T, D), loss[0, 0]
